# baseline (speedup 1.0000x reference)
import jax
import jax.numpy as jnp
from jax import lax
from jax.experimental import pallas as pl
from jax.experimental.pallas import tpu as pltpu
from jax.experimental.pallas import tpu_sc as plsc

N_NODES = 10000
N_EDGES = 320000
D = 128
NPAD = 10240
NC, NS, L = 2, 16, 16
DH = D // NC
EPT2 = N_EDGES // NS
K = 512
NCHUNK = EPT2 // K
TAIL = EPT2 - NCHUNK * K
CNT_HALF = (NCHUNK + 1) // 2
NPAIR = (NCHUNK + 1) // 2
RPT = NPAD // NS
RBLK = 160


def _sc_body(eidx_hbm, feat_hbm, zrows_hbm, zcnt_hbm, ones_hbm,
             out_sums, out_cnt0, out_cnt1,
             idx0, idx1, rows0, rows1, tidx, onesbuf, redin, redout,
             acc, cnt_acc, isem0, isem1, gsem0, gsem1, ssem0, ssem1):
    c = lax.axis_index("c")
    s = lax.axis_index("s")
    ebase = s * EPT2
    rslice = pl.ds(s * RPT, RPT)

    pltpu.async_copy(zrows_hbm, acc.at[rslice], ssem0)
    pltpu.async_copy(zcnt_hbm, cnt_acc.at[rslice], ssem0)
    pltpu.async_copy(ones_hbm, onesbuf, ssem1)

    idxbufs = (idx0, idx1)
    rowbufs = (rows0, rows1)
    isems = (isem0, isem1)
    gsems = (gsem0, gsem1)
    ssems = (ssem0, ssem1)
    cnt_lo = c * CNT_HALF
    cbase = c * N_NODES

    def eslice(k):
        return eidx_hbm.at[:, pl.ds(ebase + k * K, K)]

    def add_cbase(ib):
        for j in range(K // L):
            js = pl.ds(j * L, L)
            ib[1, js] = ib[1, js] + cbase

    pltpu.sync_copy(eslice(0), idx0)
    add_cbase(idx0)
    pltpu.async_copy(feat_hbm.at[idx0.at[1]], rows0, gsem0)
    pltpu.async_copy(eslice(1), idx1, isem1)

    pltpu.make_async_copy(zrows_hbm, acc.at[rslice], ssem0).wait()
    pltpu.make_async_copy(zcnt_hbm, cnt_acc.at[rslice], ssem0).wait()
    pltpu.make_async_copy(ones_hbm, onesbuf, ssem1).wait()
    plsc.subcore_barrier()

    def chunk_step(k, b):
        b1 = 1 - b
        ib, ib1 = idxbufs[b], idxbufs[b1]
        rb, rb1 = rowbufs[b], rowbufs[b1]
        in_cnt_window = (k >= cnt_lo) & (k < cnt_lo + CNT_HALF)
        pltpu.make_async_copy(feat_hbm.at[ib.at[1]], rb,
                              gsems[b]).wait()

        pltpu.async_copy(rb, acc.at[ib.at[0]], ssems[b], add=True)

        @pl.when(in_cnt_window)
        def _():
            pltpu.async_copy(onesbuf, cnt_acc.at[ib.at[0]], ssems[b],
                             add=True)

        @pl.when(k + 1 < NCHUNK)
        def _():
            pltpu.make_async_copy(eslice(k + 1), ib1, isems[b1]).wait()
            add_cbase(ib1)
            pltpu.async_copy(feat_hbm.at[ib1.at[1]], rb1, gsems[b1])

        pltpu.make_async_copy(rb, acc.at[ib.at[0]], ssems[b]).wait()

        @pl.when(in_cnt_window)
        def _():
            pltpu.make_async_copy(onesbuf, cnt_acc.at[ib.at[0]],
                                  ssems[b]).wait()

        @pl.when(k + 2 < NCHUNK)
        def _():
            pltpu.async_copy(eslice(k + 2), ib, isems[b])

    def pair_body(g, carry):
        for b in (0, 1):
            k = 2 * g + b

            @pl.when(k < NCHUNK)
            def _(k=k, b=b):
                chunk_step(k, b)

        return carry

    lax.fori_loop(0, NPAIR, pair_body, 0)

    trows = rows0.at[pl.ds(0, TAIL)]
    pltpu.sync_copy(eidx_hbm.at[:, pl.ds(ebase + NCHUNK * K, TAIL)], tidx)
    for j in range(TAIL // L):
        js = pl.ds(j * L, L)
        tidx[1, js] = tidx[1, js] + cbase
    pltpu.async_copy(feat_hbm.at[tidx.at[1]], trows, gsem0).wait()
    pltpu.sync_copy(trows, acc.at[tidx.at[0]], add=True)

    @pl.when(c == 1)
    def _():
        pltpu.sync_copy(onesbuf.at[pl.ds(0, TAIL)],
                        cnt_acc.at[tidx.at[0]], add=True)

    plsc.subcore_barrier()

    pltpu.async_copy(acc.at[rslice], out_sums.at[c, rslice], gsem0)

    lanes = lax.iota(jnp.int32, L)

    def red_blk(blk, carry):
        pltpu.sync_copy(cnt_acc.at[pl.ds(s * RPT + blk * RBLK, RBLK)], redin)

        def red_body(j, carry2):
            base = j * L
            v = redin[base]
            for t in range(1, L):
                v = jnp.where(lanes == t, redin[base + t], v)
            redout[pl.ds(blk * RBLK + base, L)] = v
            return carry2

        lax.fori_loop(0, RBLK // L, red_body, 0)
        return carry

    lax.fori_loop(0, RPT // RBLK, red_blk, 0)

    @pl.when(c == 0)
    def _():
        pltpu.sync_copy(redout, out_cnt0.at[rslice])

    @pl.when(c == 1)
    def _():
        pltpu.sync_copy(redout, out_cnt1.at[rslice])

    pltpu.make_async_copy(acc.at[rslice], out_sums.at[c, rslice],
                          gsem0).wait()


_sc_call = pl.kernel(
    _sc_body,
    out_type=[
        jax.ShapeDtypeStruct((NC, NPAD, DH), jnp.float32),
        jax.ShapeDtypeStruct((NPAD,), jnp.float32),
        jax.ShapeDtypeStruct((NPAD,), jnp.float32),
    ],
    mesh=plsc.VectorSubcoreMesh(core_axis_name="c", subcore_axis_name="s",
                                num_cores=NC, num_subcores=NS),
    compiler_params=pltpu.CompilerParams(use_tc_tiling_on_sc=False),
    scratch_types=[
        pltpu.VMEM((2, K), jnp.int32),
        pltpu.VMEM((2, K), jnp.int32),
        pltpu.VMEM((K, DH), jnp.float32),
        pltpu.VMEM((K, DH), jnp.float32),
        pltpu.VMEM((2, TAIL), jnp.int32),
        pltpu.VMEM((K, L), jnp.float32),
        pltpu.VMEM((RBLK, L), jnp.float32),
        pltpu.VMEM((RPT,), jnp.float32),
        pltpu.VMEM_SHARED((NPAD, DH), jnp.float32),
        pltpu.VMEM_SHARED((NPAD, L), jnp.float32),
        pltpu.SemaphoreType.DMA,
        pltpu.SemaphoreType.DMA,
        pltpu.SemaphoreType.DMA,
        pltpu.SemaphoreType.DMA,
        pltpu.SemaphoreType.DMA,
        pltpu.SemaphoreType.DMA,
    ],
)

RB = 1024


def _tc_body(feat_ref, w_ref, sums_ref, cnt0_ref, cnt1_ref, out_ref):
    w = w_ref[...]
    f = feat_ref[...]
    cnt = cnt0_ref[...] + cnt1_ref[...]
    h1 = jnp.dot(f, w, preferred_element_type=jnp.float32)
    h2 = (jnp.dot(sums_ref[0], w[:DH], preferred_element_type=jnp.float32)
          + jnp.dot(sums_ref[1], w[DH:], preferred_element_type=jnp.float32))
    inv = 1.0 / jnp.maximum(cnt, 1.0)
    h2 = (h2.reshape(RB // 128, 128, D) * inv[:, :, None]).reshape(RB, D)
    out_ref[:, :D] = h1
    out_ref[:, D:] = h2


def _tc_call(features, W, sums, cnt0, cnt1):
    return pl.pallas_call(
        _tc_body,
        grid=(NPAD // RB,),
        in_specs=[
            pl.BlockSpec((RB, D), lambda i: (i, 0)),
            pl.BlockSpec((D, D), lambda i: (0, 0)),
            pl.BlockSpec((NC, RB, DH), lambda i: (0, i, 0)),
            pl.BlockSpec((RB // 128, 128), lambda i: (i, 0)),
            pl.BlockSpec((RB // 128, 128), lambda i: (i, 0)),
        ],
        out_specs=pl.BlockSpec((RB, 2 * D), lambda i: (i, 0)),
        out_shape=jax.ShapeDtypeStruct((N_NODES, 2 * D), jnp.float32),
    )(features, W, sums, cnt0, cnt1)


def kernel(features, edge_index, W):
    feat2 = jnp.concatenate([features[:, :DH], features[:, DH:]], axis=0)
    zrows = jnp.zeros((RPT, DH), jnp.float32)
    zcnt = jnp.zeros((RPT, L), jnp.float32)
    ones = jnp.ones((K, L), jnp.float32)
    sums, cnt0, cnt1 = _sc_call(edge_index, feat2, zrows, zcnt, ones)
    return _tc_call(features, W, sums,
                    cnt0.reshape(NPAD // 128, 128),
                    cnt1.reshape(NPAD // 128, 128))


# speedup vs baseline: 1.0016x; 1.0016x over previous
import functools

import jax
import jax.numpy as jnp
from jax import lax
from jax.experimental import pallas as pl
from jax.experimental.pallas import tpu as pltpu
from jax.experimental.pallas import tpu_sc as plsc

N_NODES = 10000
N_EDGES = 320000
D = 128
NPAD = 10240
NC, NS, L = 2, 16, 16
DH = D // NC
EPT2 = N_EDGES // NS
K = 512
NCHUNK = EPT2 // K
TAIL = EPT2 - NCHUNK * K
CNT_HALF = (NCHUNK + 1) // 2
NPAIR = (NCHUNK + 1) // 2
RPT = NPAD // NS
RBLK = 160


def _sc_body(eidx_hbm, feat_hbm, zrows_hbm, zcnt_hbm, ones_hbm,
             out_sums, out_cnt0, out_cnt1,
             idx0, idx1, rows0, rows1, tidx, onesbuf, redin, redout,
             acc, cnt_acc, isem0, isem1, gsem0, gsem1, ssem0, ssem1):
    c = lax.axis_index("c")
    s = lax.axis_index("s")
    ebase = s * EPT2
    rslice = pl.ds(s * RPT, RPT)

    pltpu.async_copy(zrows_hbm, acc.at[rslice], ssem0)
    pltpu.async_copy(zcnt_hbm, cnt_acc.at[rslice], ssem0)
    pltpu.async_copy(ones_hbm, onesbuf, ssem1)

    idxbufs = (idx0, idx1)
    rowbufs = (rows0, rows1)
    isems = (isem0, isem1)
    gsems = (gsem0, gsem1)
    ssems = (ssem0, ssem1)
    cnt_lo = c * CNT_HALF
    cbase = c * N_NODES

    def eslice(k):
        return eidx_hbm.at[:, pl.ds(ebase + k * K, K)]

    def add_cbase(ib):
        for j in range(K // L):
            js = pl.ds(j * L, L)
            ib[1, js] = ib[1, js] + cbase

    pltpu.sync_copy(eslice(0), idx0)
    add_cbase(idx0)
    pltpu.async_copy(feat_hbm.at[idx0.at[1]], rows0, gsem0)
    pltpu.async_copy(eslice(1), idx1, isem1)

    pltpu.make_async_copy(zrows_hbm, acc.at[rslice], ssem0).wait()
    pltpu.make_async_copy(zcnt_hbm, cnt_acc.at[rslice], ssem0).wait()
    pltpu.make_async_copy(ones_hbm, onesbuf, ssem1).wait()
    plsc.subcore_barrier()

    def chunk_step(k, b):
        b1 = 1 - b
        ib, ib1 = idxbufs[b], idxbufs[b1]
        rb, rb1 = rowbufs[b], rowbufs[b1]
        in_cnt_window = (k >= cnt_lo) & (k < cnt_lo + CNT_HALF)
        pltpu.make_async_copy(feat_hbm.at[ib.at[1]], rb,
                              gsems[b]).wait()

        pltpu.async_copy(rb, acc.at[ib.at[0]], ssems[b], add=True)

        @pl.when(in_cnt_window)
        def _():
            pltpu.async_copy(onesbuf, cnt_acc.at[ib.at[0]], ssems[b],
                             add=True)

        @pl.when(k + 1 < NCHUNK)
        def _():
            pltpu.make_async_copy(eslice(k + 1), ib1, isems[b1]).wait()
            add_cbase(ib1)
            pltpu.async_copy(feat_hbm.at[ib1.at[1]], rb1, gsems[b1])

        pltpu.make_async_copy(rb, acc.at[ib.at[0]], ssems[b]).wait()

        @pl.when(in_cnt_window)
        def _():
            pltpu.make_async_copy(onesbuf, cnt_acc.at[ib.at[0]],
                                  ssems[b]).wait()

        @pl.when(k + 2 < NCHUNK)
        def _():
            pltpu.async_copy(eslice(k + 2), ib, isems[b])

    def pair_body(g, carry):
        for b in (0, 1):
            k = 2 * g + b

            @pl.when(k < NCHUNK)
            def _(k=k, b=b):
                chunk_step(k, b)

        return carry

    lax.fori_loop(0, NPAIR, pair_body, 0)

    trows = rows0.at[pl.ds(0, TAIL)]
    pltpu.sync_copy(eidx_hbm.at[:, pl.ds(ebase + NCHUNK * K, TAIL)], tidx)
    for j in range(TAIL // L):
        js = pl.ds(j * L, L)
        tidx[1, js] = tidx[1, js] + cbase
    pltpu.async_copy(feat_hbm.at[tidx.at[1]], trows, gsem0).wait()
    pltpu.sync_copy(trows, acc.at[tidx.at[0]], add=True)

    @pl.when(c == 1)
    def _():
        pltpu.sync_copy(onesbuf.at[pl.ds(0, TAIL)],
                        cnt_acc.at[tidx.at[0]], add=True)

    plsc.subcore_barrier()

    pltpu.async_copy(acc.at[rslice], out_sums.at[c, rslice], gsem0)

    lanes = lax.iota(jnp.int32, L)

    def red_blk(blk, carry):
        pltpu.sync_copy(cnt_acc.at[pl.ds(s * RPT + blk * RBLK, RBLK)], redin)

        def red_body(j, carry2):
            base = j * L
            v = redin[base]
            for t in range(1, L):
                v = jnp.where(lanes == t, redin[base + t], v)
            redout[pl.ds(blk * RBLK + base, L)] = v
            return carry2

        lax.fori_loop(0, RBLK // L, red_body, 0)
        return carry

    lax.fori_loop(0, RPT // RBLK, red_blk, 0)

    @pl.when(c == 0)
    def _():
        pltpu.sync_copy(redout, out_cnt0.at[rslice])

    @pl.when(c == 1)
    def _():
        pltpu.sync_copy(redout, out_cnt1.at[rslice])

    pltpu.make_async_copy(acc.at[rslice], out_sums.at[c, rslice],
                          gsem0).wait()


@functools.lru_cache(maxsize=1)
def _get_sc_call():
  return pl.kernel(
    _sc_body,
    out_type=[
        jax.ShapeDtypeStruct((NC, NPAD, DH), jnp.float32),
        jax.ShapeDtypeStruct((NPAD,), jnp.float32),
        jax.ShapeDtypeStruct((NPAD,), jnp.float32),
    ],
    mesh=plsc.VectorSubcoreMesh(core_axis_name="c", subcore_axis_name="s",
                                num_cores=NC, num_subcores=NS),
    compiler_params=pltpu.CompilerParams(use_tc_tiling_on_sc=False),
    scratch_types=[
        pltpu.VMEM((2, K), jnp.int32),
        pltpu.VMEM((2, K), jnp.int32),
        pltpu.VMEM((K, DH), jnp.float32),
        pltpu.VMEM((K, DH), jnp.float32),
        pltpu.VMEM((2, TAIL), jnp.int32),
        pltpu.VMEM((K, L), jnp.float32),
        pltpu.VMEM((RBLK, L), jnp.float32),
        pltpu.VMEM((RPT,), jnp.float32),
        pltpu.VMEM_SHARED((NPAD, DH), jnp.float32),
        pltpu.VMEM_SHARED((NPAD, L), jnp.float32),
        pltpu.SemaphoreType.DMA,
        pltpu.SemaphoreType.DMA,
        pltpu.SemaphoreType.DMA,
        pltpu.SemaphoreType.DMA,
        pltpu.SemaphoreType.DMA,
        pltpu.SemaphoreType.DMA,
    ],
  )

RB = 1024


def _tc_body(feat_ref, w_ref, sums_ref, cnt0_ref, cnt1_ref, out_ref):
    w = w_ref[...]
    f = feat_ref[...]
    cnt = cnt0_ref[...] + cnt1_ref[...]
    h1 = jnp.dot(f, w, preferred_element_type=jnp.float32)
    h2 = (jnp.dot(sums_ref[0], w[:DH], preferred_element_type=jnp.float32)
          + jnp.dot(sums_ref[1], w[DH:], preferred_element_type=jnp.float32))
    inv = 1.0 / jnp.maximum(cnt, 1.0)
    h2 = (h2.reshape(RB // 128, 128, D) * inv[:, :, None]).reshape(RB, D)
    out_ref[:, :D] = h1
    out_ref[:, D:] = h2


def _tc_call(features, W, sums, cnt0, cnt1):
    return pl.pallas_call(
        _tc_body,
        grid=(NPAD // RB,),
        in_specs=[
            pl.BlockSpec((RB, D), lambda i: (i, 0)),
            pl.BlockSpec((D, D), lambda i: (0, 0)),
            pl.BlockSpec((NC, RB, DH), lambda i: (0, i, 0)),
            pl.BlockSpec((RB // 128, 128), lambda i: (i, 0)),
            pl.BlockSpec((RB // 128, 128), lambda i: (i, 0)),
        ],
        out_specs=pl.BlockSpec((RB, 2 * D), lambda i: (i, 0)),
        out_shape=jax.ShapeDtypeStruct((N_NODES, 2 * D), jnp.float32),
    )(features, W, sums, cnt0, cnt1)


def kernel(features, edge_index, W):
    feat2 = jnp.concatenate([features[:, :DH], features[:, DH:]], axis=0)
    zrows = jnp.zeros((RPT, DH), jnp.float32)
    zcnt = jnp.zeros((RPT, L), jnp.float32)
    ones = jnp.ones((K, L), jnp.float32)
    sums, cnt0, cnt1 = _get_sc_call()(edge_index, feat2, zrows, zcnt, ones)
    return _tc_call(features, W, sums,
                    cnt0.reshape(NPAD // 128, 128),
                    cnt1.reshape(NPAD // 128, 128))
